# baseline (speedup 1.0000x reference)
import functools

import jax
import jax.numpy as jnp
from jax import lax
from jax.experimental import pallas as pl
from jax.experimental.pallas import tpu as pltpu
from jax.experimental.pallas import tpu_sc as plsc

B, S, D = 4, 4096, 4096
NUM_EXPERTS = 64
K = 8
ROWS = B * S
BLK = 512
PADE = 72

NC, NS, L = 2, 16, 16
NW = NC * NS
RPW = ROWS // NW
GROUPS = RPW // L


def _router_block(x_ref, w_ref, probs_ref, probs_t_ref):
    s = jnp.dot(x_ref[...], w_ref[...], preferred_element_type=jnp.float32)
    m = jnp.max(s, axis=-1, keepdims=True)
    e = jnp.exp(s - m)
    p = e / jnp.sum(e, axis=-1, keepdims=True)
    probs_ref[...] = p
    work = p
    for _ in range(K):
        t = jnp.max(work, axis=-1, keepdims=True)
        work = jnp.where(work == t, -jnp.inf, work)
    probs_t_ref[0:NUM_EXPERTS, :] = p.T
    probs_t_ref[NUM_EXPERTS:NUM_EXPERTS + 1, :] = t.T
    probs_t_ref[NUM_EXPERTS + 1:PADE, :] = jnp.zeros((PADE - NUM_EXPERTS - 1, BLK), jnp.float32)


def _tc_router(xf, expert_embs):
    return pl.pallas_call(
        _router_block,
        grid=(ROWS // BLK,),
        in_specs=[
            pl.BlockSpec((BLK, D), lambda i: (i, 0)),
            pl.BlockSpec((D, NUM_EXPERTS), lambda i: (0, 0)),
        ],
        out_specs=[
            pl.BlockSpec((BLK, NUM_EXPERTS), lambda i: (i, 0)),
            pl.BlockSpec((PADE, BLK), lambda i: (0, i)),
        ],
        out_shape=[
            jax.ShapeDtypeStruct((ROWS, NUM_EXPERTS), jnp.float32),
            jax.ShapeDtypeStruct((PADE, ROWS), jnp.float32),
        ],
    )(xf, expert_embs)


def _sc_mask_body(pt_hbm, out_hbm, in_v, out_v):
    wid = lax.axis_index("s") * NC + lax.axis_index("c")
    base = wid * RPW
    pltpu.sync_copy(pt_hbm.at[:, pl.ds(base, RPW)], in_v)

    @plsc.parallel_loop(0, GROUPS, 1, unroll=2)
    def group(g):
        lr = g * L
        t = in_v[NUM_EXPERTS, pl.ds(lr, L)]
        for e in range(NUM_EXPERTS):
            v = in_v[e, pl.ds(lr, L)]
            out_v[e, pl.ds(lr, L)] = jnp.where(v >= t, v, 0.0)

    pltpu.sync_copy(out_v, out_hbm.at[:, pl.ds(base, RPW)])


@functools.partial(
    pl.kernel,
    mesh=plsc.VectorSubcoreMesh(core_axis_name="c", subcore_axis_name="s"),
    compiler_params=pltpu.CompilerParams(needs_layout_passes=False),
    out_type=jax.ShapeDtypeStruct((NUM_EXPERTS, ROWS), jnp.float32),
    scratch_types=[
        pltpu.VMEM((PADE, RPW), jnp.float32),
        pltpu.VMEM((NUM_EXPERTS, RPW), jnp.float32),
    ],
)
def _sc_mask(pt_hbm, out_hbm, in_v, out_v):
    _sc_mask_body(pt_hbm, out_hbm, in_v, out_v)


def kernel(x, expert_embs):
    xf = x.reshape(ROWS, D)
    probs, probs_t = _tc_router(xf, expert_embs)
    masks_t = _sc_mask(probs_t)
    experts_masks = masks_t.reshape(NUM_EXPERTS, B, S, 1)
    aux_loss = jnp.zeros((), jnp.float32)
    return (experts_masks, aux_loss, probs)


# speedup vs baseline: 1.0717x; 1.0717x over previous
import functools

import jax
import jax.numpy as jnp
from jax import lax
from jax.experimental import pallas as pl
from jax.experimental.pallas import tpu as pltpu
from jax.experimental.pallas import tpu_sc as plsc

B, S, D = 4, 4096, 4096
NUM_EXPERTS = 64
K = 8
ROWS = B * S
BLK = 1024
PADE = 72

NC, NS, L = 2, 16, 16
NW = NC * NS
RPW = ROWS // NW
GROUPS = RPW // L


def _router_block(x_ref, w_ref, probs_ref, probs_t_ref):
    s = jnp.dot(x_ref[...], w_ref[...], preferred_element_type=jnp.float32)
    m = jnp.max(s, axis=-1, keepdims=True)
    e = jnp.exp(s - m)
    p = e / jnp.sum(e, axis=-1, keepdims=True)
    probs_ref[...] = p
    work = p
    for _ in range(K):
        t = jnp.max(work, axis=-1, keepdims=True)
        work = jnp.where(work == t, -jnp.inf, work)
    probs_t_ref[0:NUM_EXPERTS, :] = p.T
    probs_t_ref[NUM_EXPERTS:NUM_EXPERTS + 1, :] = t.T
    probs_t_ref[NUM_EXPERTS + 1:PADE, :] = jnp.zeros((PADE - NUM_EXPERTS - 1, BLK), jnp.float32)


def _tc_router(xf, expert_embs):
    return pl.pallas_call(
        _router_block,
        grid=(ROWS // BLK,),
        in_specs=[
            pl.BlockSpec((BLK, D), lambda i: (i, 0)),
            pl.BlockSpec((D, NUM_EXPERTS), lambda i: (0, 0)),
        ],
        out_specs=[
            pl.BlockSpec((BLK, NUM_EXPERTS), lambda i: (i, 0)),
            pl.BlockSpec((PADE, BLK), lambda i: (0, i)),
        ],
        out_shape=[
            jax.ShapeDtypeStruct((ROWS, NUM_EXPERTS), jnp.float32),
            jax.ShapeDtypeStruct((PADE, ROWS), jnp.float32),
        ],
    )(xf, expert_embs)


def _sc_mask_body(pt_hbm, out_hbm, in_v, out_v):
    wid = lax.axis_index("s") * NC + lax.axis_index("c")
    base = wid * RPW
    pltpu.sync_copy(pt_hbm.at[:, pl.ds(base, RPW)], in_v)

    @plsc.parallel_loop(0, GROUPS, 1, unroll=2)
    def group(g):
        lr = g * L
        t = in_v[NUM_EXPERTS, pl.ds(lr, L)]
        for e in range(NUM_EXPERTS):
            v = in_v[e, pl.ds(lr, L)]
            out_v[e, pl.ds(lr, L)] = jnp.where(v >= t, v, 0.0)

    pltpu.sync_copy(out_v, out_hbm.at[:, pl.ds(base, RPW)])


@functools.partial(
    pl.kernel,
    mesh=plsc.VectorSubcoreMesh(core_axis_name="c", subcore_axis_name="s"),
    compiler_params=pltpu.CompilerParams(needs_layout_passes=False),
    out_type=jax.ShapeDtypeStruct((NUM_EXPERTS, ROWS), jnp.float32),
    scratch_types=[
        pltpu.VMEM((PADE, RPW), jnp.float32),
        pltpu.VMEM((NUM_EXPERTS, RPW), jnp.float32),
    ],
)
def _sc_mask(pt_hbm, out_hbm, in_v, out_v):
    _sc_mask_body(pt_hbm, out_hbm, in_v, out_v)


def kernel(x, expert_embs):
    xf = x.reshape(ROWS, D)
    probs, probs_t = _tc_router(xf, expert_embs)
    masks_t = _sc_mask(probs_t)
    experts_masks = masks_t.reshape(NUM_EXPERTS, B, S, 1)
    aux_loss = jnp.zeros((), jnp.float32)
    return (experts_masks, aux_loss, probs)
